# baseline (speedup 1.0000x reference)
import dataclasses
import functools

import jax
import jax.numpy as jnp
from jax import lax
from jax.experimental import pallas as pl
from jax.experimental.pallas import tpu as pltpu
from jax.experimental.pallas import tpu_sc as plsc

NUM_WORKERS = 32
_LANES = 16

_cp = pltpu.CompilerParams()
if "needs_layout_passes" in pltpu.CompilerParams.__dataclass_fields__:
    _cp = dataclasses.replace(_cp, needs_layout_passes=False)
if "use_tc_tiling_on_sc" in pltpu.CompilerParams.__dataclass_fields__:
    _cp = dataclasses.replace(_cp, use_tc_tiling_on_sc=False)

_MESH = plsc.VectorSubcoreMesh(core_axis_name="c", subcore_axis_name="s")

_PREC = jax.lax.Precision.HIGHEST
_PREC_SMALL = jax.lax.Precision.HIGHEST


def _zero_fill(ref, total, unroll):
    zero16 = jnp.zeros((_LANES,), jnp.float32)
    step = unroll * _LANES
    assert total % step == 0

    @pl.loop(0, total, step=step)
    def _(i):
        for u in range(unroll):
            ref[pl.ds(i + u * _LANES, _LANES)] = zero16


@functools.lru_cache(maxsize=None)
def _make_deg_kernel(n_nodes: int, n_edges: int):
    ep = n_edges // NUM_WORKERS
    assert n_edges % NUM_WORKERS == 0 and ep % _LANES == 0
    ep_main = (ep // 32) * 32

    @functools.partial(
        pl.kernel,
        out_type=[
            jax.ShapeDtypeStruct((NUM_WORKERS, n_nodes), jnp.float32),
            jax.ShapeDtypeStruct((NUM_WORKERS, 2 * ep), jnp.int32),
        ],
        mesh=_MESH,
        scratch_types=[
            pltpu.VMEM((ep,), jnp.int32),
            pltpu.VMEM((ep,), jnp.int32),
            pltpu.VMEM((n_nodes,), jnp.float32),
            pltpu.SemaphoreType.DMA,
            pltpu.SemaphoreType.DMA,
        ],
        compiler_params=_cp,
    )
    def deg_kernel(ei_hbm, out_hbm, edges_hbm, src_v, dst_v, deg_v, sem_s, sem_d):
        wid = lax.axis_index("s") * 2 + lax.axis_index("c")
        cp_d = pltpu.async_copy(ei_hbm.at[1, pl.ds(wid * ep, ep)], dst_v, sem_d)
        cp_s = pltpu.async_copy(ei_hbm.at[0, pl.ds(wid * ep, ep)], src_v, sem_s)
        _zero_fill(deg_v, n_nodes, 5)
        cp_d.wait()
        cp_s.wait()
        cp_so = pltpu.async_copy(src_v, edges_hbm.at[wid, pl.ds(0, ep)], sem_s)

        ones16 = jnp.ones((_LANES,), jnp.float32)

        @pl.loop(0, ep_main, step=2 * _LANES)
        def _(i):
            d0 = dst_v[pl.ds(i, _LANES)]
            d1 = dst_v[pl.ds(i + _LANES, _LANES)]
            plsc.addupdate_scatter(deg_v, [d0], ones16)
            plsc.addupdate_scatter(deg_v, [d1], ones16)

        for i in range(ep_main, ep, _LANES):
            plsc.addupdate_scatter(deg_v, [dst_v[pl.ds(i, _LANES)]], ones16)

        cp_so.wait()
        pltpu.sync_copy(dst_v, edges_hbm.at[wid, pl.ds(ep, ep)])
        pltpu.sync_copy(deg_v, out_hbm.at[wid])

    return deg_kernel


@functools.lru_cache(maxsize=None)
def _make_agg_kernel(n_nodes: int, n_edges: int, w: int):
    ep = n_edges // NUM_WORKERS
    tbl = w * n_nodes
    assert n_edges % NUM_WORKERS == 0 and ep % _LANES == 0
    assert tbl % (10 * _LANES) == 0
    ep_main = (ep // 32) * 32

    @functools.partial(
        pl.kernel,
        out_type=jax.ShapeDtypeStruct((NUM_WORKERS, tbl), jnp.float32),
        mesh=_MESH,
        scratch_types=[
            pltpu.VMEM((ep,), jnp.int32),
            pltpu.VMEM((ep,), jnp.int32),
            pltpu.VMEM((tbl,), jnp.float32),
            pltpu.VMEM((tbl,), jnp.float32),
            pltpu.SemaphoreType.DMA,
            pltpu.SemaphoreType.DMA,
            pltpu.SemaphoreType.DMA,
        ],
        compiler_params=_cp,
    )
    def agg_kernel(z_hbm, edges_hbm, out_hbm,
                   src_v, dst_v, z_v, acc_v, sem_s, sem_d, sem_z):
        wid = lax.axis_index("s") * 2 + lax.axis_index("c")
        cp_s = pltpu.async_copy(edges_hbm.at[wid, pl.ds(0, ep)], src_v, sem_s)
        cp_d = pltpu.async_copy(edges_hbm.at[wid, pl.ds(ep, ep)], dst_v, sem_d)
        cp_z = pltpu.async_copy(z_hbm, z_v, sem_z)
        _zero_fill(acc_v, tbl, 10)
        cp_s.wait()
        cp_d.wait()
        cp_z.wait()

        def do_group(s, d):
            vals = []
            for c in range(w):
                si = (s + c * n_nodes) if c else s
                vals.append((plsc.load_gather(z_v, [si]),
                             (d + c * n_nodes) if c else d))
            for v, di in vals:
                plsc.addupdate_scatter(acc_v, [di], v)

        @pl.loop(0, ep_main, step=2 * _LANES)
        def _(i):
            s0 = src_v[pl.ds(i, _LANES)]
            d0 = dst_v[pl.ds(i, _LANES)]
            s1 = src_v[pl.ds(i + _LANES, _LANES)]
            d1 = dst_v[pl.ds(i + _LANES, _LANES)]
            do_group(s0, d0)
            do_group(s1, d1)

        for i in range(ep_main, ep, _LANES):
            do_group(src_v[pl.ds(i, _LANES)], dst_v[pl.ds(i, _LANES)])

        pltpu.sync_copy(acc_v, out_hbm.at[wid])

    return agg_kernel


def _rows2d(flat, w, n):
    return jnp.concatenate([flat[c * n:(c + 1) * n][None, :] for c in range(w)],
                           axis=0)


def _tc1_body(deg_parts_ref, x_ref, w1t_ref, dinv_ref, z1_ref):
    n = x_ref.shape[0]
    w1 = w1t_ref.shape[0]
    deg = jnp.sum(deg_parts_ref[...], axis=0) + 1.0
    dinv = lax.rsqrt(deg)
    dinv_ref[...] = dinv
    xwt = lax.dot_general(w1t_ref[...], x_ref[...],
                          (((1,), (1,)), ((), ())),
                          precision=_PREC,
                          preferred_element_type=jnp.float32)
    for c in range(w1):
        z1_ref[pl.ds(c * n, n)] = xwt[c] * dinv


def _tc_layer_body(parts_ref, z_ref, dinv_ref, wt_ref, bcol_ref, zout_ref):
    w_out, w = wt_ref.shape
    n = dinv_ref.shape[0]
    flat = jnp.sum(parts_ref[...], axis=0)
    acc = _rows2d(flat + z_ref[...], w, n)
    dinv = dinv_ref[...][None, :]
    h = jnp.maximum(dinv * acc + bcol_ref[...], 0.0)
    zout = lax.dot_general(wt_ref[...], h,
                           (((1,), (0,)), ((), ())),
                           precision=_PREC_SMALL,
                           preferred_element_type=jnp.float32) * dinv
    for c in range(w_out):
        zout_ref[pl.ds(c * n, n)] = zout[c]


def _tc_final_body(parts_ref, z_ref, dinv_ref, wcx_ref, b3col_ref, bcrow_ref,
                   h_ref, out_ref):
    w = b3col_ref.shape[0]
    n = dinv_ref.shape[0]
    flat = jnp.sum(parts_ref[...], axis=0)
    acc = _rows2d(flat + z_ref[...], w, n)
    dinv = dinv_ref[...][None, :]
    h = jnp.maximum(dinv * acc + b3col_ref[...], 0.0)
    both = lax.dot_general(h, wcx_ref[...],
                           (((0,), (0,)), ((), ())),
                           precision=_PREC_SMALL,
                           preferred_element_type=jnp.float32)
    h_ref[...] = both[:, :w]
    out_ref[...] = both[:, w:] + bcrow_ref[...]


def _f32(shape):
    return jax.ShapeDtypeStruct(shape, jnp.float32)


def kernel(x, edge_index, W1, b1, W2, b2, W3, b3, Wc, bc):
    n, _ = x.shape
    e = edge_index.shape[1]
    w1 = W1.shape[1]
    w2 = W2.shape[1]
    w3 = W3.shape[1]
    c_out = Wc.shape[1]

    ei = edge_index.astype(jnp.int32)

    deg_parts, edges = _make_deg_kernel(n, e)(ei)

    dinv, z1 = pl.pallas_call(
        _tc1_body,
        out_shape=[_f32((n,)), _f32((w1 * n,))],
    )(deg_parts, x, W1.T)

    p1 = _make_agg_kernel(n, e, w1)(z1, edges)
    z2 = pl.pallas_call(
        _tc_layer_body,
        out_shape=_f32((w2 * n,)),
    )(p1, z1, dinv, W2.T, b1[:, None])

    p2 = _make_agg_kernel(n, e, w2)(z2, edges)
    z3 = pl.pallas_call(
        _tc_layer_body,
        out_shape=_f32((w3 * n,)),
    )(p2, z2, dinv, W3.T, b2[:, None])

    p3 = _make_agg_kernel(n, e, w3)(z3, edges)
    wcx = jnp.concatenate([jnp.eye(w3, dtype=jnp.float32), Wc], axis=1)
    h, out = pl.pallas_call(
        _tc_final_body,
        out_shape=[_f32((n, w3)), _f32((n, c_out))],
    )(p3, z3, dinv, wcx, b3[:, None], bc[None, :])

    return (out, h)


# speedup vs baseline: 1.0626x; 1.0626x over previous
import dataclasses
import functools

import jax
import jax.numpy as jnp
from jax import lax
from jax.experimental import pallas as pl
from jax.experimental.pallas import tpu as pltpu
from jax.experimental.pallas import tpu_sc as plsc

NUM_WORKERS = 32
_LANES = 16

_cp = pltpu.CompilerParams()
if "needs_layout_passes" in pltpu.CompilerParams.__dataclass_fields__:
    _cp = dataclasses.replace(_cp, needs_layout_passes=False)
if "use_tc_tiling_on_sc" in pltpu.CompilerParams.__dataclass_fields__:
    _cp = dataclasses.replace(_cp, use_tc_tiling_on_sc=False)

_MESH = plsc.VectorSubcoreMesh(core_axis_name="c", subcore_axis_name="s")

_PREC = jax.lax.Precision.HIGHEST
_PREC_SMALL = jax.lax.Precision.DEFAULT


def _zero_fill(ref, total, unroll):
    zero16 = jnp.zeros((_LANES,), jnp.float32)
    step = unroll * _LANES
    assert total % step == 0

    @pl.loop(0, total, step=step)
    def _(i):
        for u in range(unroll):
            ref[pl.ds(i + u * _LANES, _LANES)] = zero16


@functools.lru_cache(maxsize=None)
def _make_deg_kernel(n_nodes: int, n_edges: int):
    ep = n_edges // NUM_WORKERS
    assert n_edges % NUM_WORKERS == 0 and ep % _LANES == 0
    ep_main = (ep // 32) * 32

    @functools.partial(
        pl.kernel,
        out_type=[
            jax.ShapeDtypeStruct((NUM_WORKERS, n_nodes), jnp.float32),
            jax.ShapeDtypeStruct((NUM_WORKERS, 2 * ep), jnp.int32),
        ],
        mesh=_MESH,
        scratch_types=[
            pltpu.VMEM((ep,), jnp.int32),
            pltpu.VMEM((ep,), jnp.int32),
            pltpu.VMEM((n_nodes,), jnp.float32),
            pltpu.SemaphoreType.DMA,
            pltpu.SemaphoreType.DMA,
        ],
        compiler_params=_cp,
    )
    def deg_kernel(ei_hbm, out_hbm, edges_hbm, src_v, dst_v, deg_v, sem_s, sem_d):
        wid = lax.axis_index("s") * 2 + lax.axis_index("c")
        cp_d = pltpu.async_copy(ei_hbm.at[1, pl.ds(wid * ep, ep)], dst_v, sem_d)
        cp_s = pltpu.async_copy(ei_hbm.at[0, pl.ds(wid * ep, ep)], src_v, sem_s)
        _zero_fill(deg_v, n_nodes, 5)
        cp_d.wait()
        cp_s.wait()
        cp_so = pltpu.async_copy(src_v, edges_hbm.at[wid, pl.ds(0, ep)], sem_s)

        ones16 = jnp.ones((_LANES,), jnp.float32)

        @pl.loop(0, ep_main, step=2 * _LANES)
        def _(i):
            d0 = dst_v[pl.ds(i, _LANES)]
            d1 = dst_v[pl.ds(i + _LANES, _LANES)]
            plsc.addupdate_scatter(deg_v, [d0], ones16)
            plsc.addupdate_scatter(deg_v, [d1], ones16)

        for i in range(ep_main, ep, _LANES):
            plsc.addupdate_scatter(deg_v, [dst_v[pl.ds(i, _LANES)]], ones16)

        cp_so.wait()
        pltpu.sync_copy(dst_v, edges_hbm.at[wid, pl.ds(ep, ep)])
        pltpu.sync_copy(deg_v, out_hbm.at[wid])

    return deg_kernel


@functools.lru_cache(maxsize=None)
def _make_agg_kernel(n_nodes: int, n_edges: int, w: int):
    ep = n_edges // NUM_WORKERS
    tbl = w * n_nodes
    assert n_edges % NUM_WORKERS == 0 and ep % _LANES == 0
    assert tbl % (10 * _LANES) == 0
    ep_main = (ep // 32) * 32

    @functools.partial(
        pl.kernel,
        out_type=jax.ShapeDtypeStruct((NUM_WORKERS, tbl), jnp.float32),
        mesh=_MESH,
        scratch_types=[
            pltpu.VMEM((ep,), jnp.int32),
            pltpu.VMEM((ep,), jnp.int32),
            pltpu.VMEM((tbl,), jnp.float32),
            pltpu.VMEM((tbl,), jnp.float32),
            pltpu.SemaphoreType.DMA,
            pltpu.SemaphoreType.DMA,
            pltpu.SemaphoreType.DMA,
        ],
        compiler_params=_cp,
    )
    def agg_kernel(z_hbm, edges_hbm, out_hbm,
                   src_v, dst_v, z_v, acc_v, sem_s, sem_d, sem_z):
        wid = lax.axis_index("s") * 2 + lax.axis_index("c")
        cp_s = pltpu.async_copy(edges_hbm.at[wid, pl.ds(0, ep)], src_v, sem_s)
        cp_d = pltpu.async_copy(edges_hbm.at[wid, pl.ds(ep, ep)], dst_v, sem_d)
        cp_z = pltpu.async_copy(z_hbm, z_v, sem_z)
        _zero_fill(acc_v, tbl, 10)
        cp_s.wait()
        cp_d.wait()
        cp_z.wait()

        def do_group(s, d):
            vals = []
            for c in range(w):
                si = (s + c * n_nodes) if c else s
                vals.append((plsc.load_gather(z_v, [si]),
                             (d + c * n_nodes) if c else d))
            for v, di in vals:
                plsc.addupdate_scatter(acc_v, [di], v)

        @pl.loop(0, ep_main, step=2 * _LANES)
        def _(i):
            s0 = src_v[pl.ds(i, _LANES)]
            d0 = dst_v[pl.ds(i, _LANES)]
            s1 = src_v[pl.ds(i + _LANES, _LANES)]
            d1 = dst_v[pl.ds(i + _LANES, _LANES)]
            do_group(s0, d0)
            do_group(s1, d1)

        for i in range(ep_main, ep, _LANES):
            do_group(src_v[pl.ds(i, _LANES)], dst_v[pl.ds(i, _LANES)])

        pltpu.sync_copy(acc_v, out_hbm.at[wid])

    return agg_kernel


def _rows2d(flat, w, n):
    return jnp.concatenate([flat[c * n:(c + 1) * n][None, :] for c in range(w)],
                           axis=0)


def _tc0_body(x_ref, w1t_ref, xw1_ref):
    n = x_ref.shape[0]
    w1 = w1t_ref.shape[0]
    xwt = lax.dot_general(w1t_ref[...], x_ref[...],
                          (((1,), (1,)), ((), ())),
                          precision=_PREC,
                          preferred_element_type=jnp.float32)
    for c in range(w1):
        xw1_ref[pl.ds(c * n, n)] = xwt[c]


def _tc1_body(deg_parts_ref, xw1_ref, dinv_ref, z1_ref):
    n = dinv_ref.shape[0]
    w1 = xw1_ref.shape[0] // n
    deg = jnp.sum(deg_parts_ref[...], axis=0) + 1.0
    dinv = lax.rsqrt(deg)
    dinv_ref[...] = dinv
    for c in range(w1):
        z1_ref[pl.ds(c * n, n)] = xw1_ref[pl.ds(c * n, n)] * dinv


def _tc_layer_body(parts_ref, z_ref, dinv_ref, wt_ref, bcol_ref, zout_ref):
    w_out, w = wt_ref.shape
    n = dinv_ref.shape[0]
    flat = jnp.sum(parts_ref[...], axis=0)
    acc = _rows2d(flat + z_ref[...], w, n)
    dinv = dinv_ref[...][None, :]
    h = jnp.maximum(dinv * acc + bcol_ref[...], 0.0)
    zout = lax.dot_general(wt_ref[...], h,
                           (((1,), (0,)), ((), ())),
                           precision=_PREC_SMALL,
                           preferred_element_type=jnp.float32) * dinv
    for c in range(w_out):
        zout_ref[pl.ds(c * n, n)] = zout[c]


def _tc_final_body(parts_ref, z_ref, dinv_ref, wcx_ref, b3col_ref, bcrow_ref,
                   h_ref, out_ref):
    w = b3col_ref.shape[0]
    n = dinv_ref.shape[0]
    flat = jnp.sum(parts_ref[...], axis=0)
    acc = _rows2d(flat + z_ref[...], w, n)
    dinv = dinv_ref[...][None, :]
    h = jnp.maximum(dinv * acc + b3col_ref[...], 0.0)
    both = lax.dot_general(h, wcx_ref[...],
                           (((0,), (0,)), ((), ())),
                           precision=_PREC_SMALL,
                           preferred_element_type=jnp.float32)
    h_ref[...] = both[:, :w]
    out_ref[...] = both[:, w:] + bcrow_ref[...]


def _f32(shape):
    return jax.ShapeDtypeStruct(shape, jnp.float32)


def kernel(x, edge_index, W1, b1, W2, b2, W3, b3, Wc, bc):
    n, _ = x.shape
    e = edge_index.shape[1]
    w1 = W1.shape[1]
    w2 = W2.shape[1]
    w3 = W3.shape[1]
    c_out = Wc.shape[1]

    ei = edge_index.astype(jnp.int32)

    deg_parts, edges = _make_deg_kernel(n, e)(ei)

    xw1 = pl.pallas_call(_tc0_body, out_shape=_f32((w1 * n,)))(x, W1.T)

    dinv, z1 = pl.pallas_call(
        _tc1_body,
        out_shape=[_f32((n,)), _f32((w1 * n,))],
    )(deg_parts, xw1)

    p1 = _make_agg_kernel(n, e, w1)(z1, edges)
    z2 = pl.pallas_call(
        _tc_layer_body,
        out_shape=_f32((w2 * n,)),
    )(p1, z1, dinv, W2.T, b1[:, None])

    p2 = _make_agg_kernel(n, e, w2)(z2, edges)
    z3 = pl.pallas_call(
        _tc_layer_body,
        out_shape=_f32((w3 * n,)),
    )(p2, z2, dinv, W3.T, b2[:, None])

    p3 = _make_agg_kernel(n, e, w3)(z3, edges)
    wcx = jnp.concatenate([jnp.eye(w3, dtype=jnp.float32), Wc], axis=1)
    h, out = pl.pallas_call(
        _tc_final_body,
        out_shape=[_f32((n, w3)), _f32((n, c_out))],
    )(p3, z3, dinv, wcx, b3[:, None], bc[None, :])

    return (out, h)


# speedup vs baseline: 1.2614x; 1.1870x over previous
import dataclasses
import functools

import jax
import jax.numpy as jnp
from jax import lax
from jax.experimental import pallas as pl
from jax.experimental.pallas import tpu as pltpu
from jax.experimental.pallas import tpu_sc as plsc

NUM_WORKERS = 32
_LANES = 16

_cp = pltpu.CompilerParams()
if "needs_layout_passes" in pltpu.CompilerParams.__dataclass_fields__:
    _cp = dataclasses.replace(_cp, needs_layout_passes=False)
if "use_tc_tiling_on_sc" in pltpu.CompilerParams.__dataclass_fields__:
    _cp = dataclasses.replace(_cp, use_tc_tiling_on_sc=False)

_MESH = plsc.VectorSubcoreMesh(core_axis_name="c", subcore_axis_name="s")

_PREC = jax.lax.Precision.HIGHEST
_PREC_SMALL = jax.lax.Precision.DEFAULT


def _zero_fill(ref, total, unroll):
    zero16 = jnp.zeros((_LANES,), jnp.float32)
    step = unroll * _LANES
    assert total % step == 0

    @pl.loop(0, total, step=step)
    def _(i):
        for u in range(unroll):
            ref[pl.ds(i + u * _LANES, _LANES)] = zero16


def _pad128(x):
    return ((x + 127) // 128) * 128


@functools.lru_cache(maxsize=None)
def _make_deg_kernel(n_nodes: int, n_edges: int):
    ep = n_edges // NUM_WORKERS
    npad = _pad128(n_nodes)
    assert n_edges % NUM_WORKERS == 0 and ep % _LANES == 0
    ep_main = (ep // 32) * 32

    @functools.partial(
        pl.kernel,
        out_type=[
            jax.ShapeDtypeStruct((NUM_WORKERS * npad,), jnp.float32),
            jax.ShapeDtypeStruct((2 * n_edges,), jnp.int32),
        ],
        mesh=_MESH,
        scratch_types=[
            pltpu.VMEM((ep,), jnp.int32),
            pltpu.VMEM((ep,), jnp.int32),
            pltpu.VMEM((n_nodes,), jnp.float32),
            pltpu.SemaphoreType.DMA,
            pltpu.SemaphoreType.DMA,
        ],
        compiler_params=_cp,
    )
    def deg_kernel(ei_hbm, out_hbm, edges_hbm, src_v, dst_v, deg_v, sem_s, sem_d):
        wid = lax.axis_index("s") * 2 + lax.axis_index("c")
        cp_d = pltpu.async_copy(ei_hbm.at[1, pl.ds(wid * ep, ep)], dst_v, sem_d)
        cp_s = pltpu.async_copy(ei_hbm.at[0, pl.ds(wid * ep, ep)], src_v, sem_s)
        _zero_fill(deg_v, n_nodes, 5)
        cp_d.wait()
        cp_s.wait()
        cp_so = pltpu.async_copy(src_v, edges_hbm.at[pl.ds(wid * 2 * ep, ep)],
                                 sem_s)

        ones16 = jnp.ones((_LANES,), jnp.float32)

        @pl.loop(0, ep_main, step=2 * _LANES)
        def _(i):
            d0 = dst_v[pl.ds(i, _LANES)]
            d1 = dst_v[pl.ds(i + _LANES, _LANES)]
            plsc.addupdate_scatter(deg_v, [d0], ones16)
            plsc.addupdate_scatter(deg_v, [d1], ones16)

        for i in range(ep_main, ep, _LANES):
            plsc.addupdate_scatter(deg_v, [dst_v[pl.ds(i, _LANES)]], ones16)

        cp_so.wait()
        pltpu.sync_copy(dst_v, edges_hbm.at[pl.ds(wid * 2 * ep + ep, ep)])
        pltpu.sync_copy(deg_v, out_hbm.at[pl.ds(wid * npad, n_nodes)])

    return deg_kernel


@functools.lru_cache(maxsize=None)
def _make_agg_kernel(n_nodes: int, n_edges: int, w: int):
    ep = n_edges // NUM_WORKERS
    tbl = w * n_nodes
    tblp = _pad128(tbl)
    assert n_edges % NUM_WORKERS == 0 and ep % _LANES == 0
    assert tbl % (10 * _LANES) == 0
    ep_main = (ep // 32) * 32

    @functools.partial(
        pl.kernel,
        out_type=jax.ShapeDtypeStruct((NUM_WORKERS * tblp,), jnp.float32),
        mesh=_MESH,
        scratch_types=[
            pltpu.VMEM((ep,), jnp.int32),
            pltpu.VMEM((ep,), jnp.int32),
            pltpu.VMEM((tbl,), jnp.float32),
            pltpu.VMEM((tbl,), jnp.float32),
            pltpu.SemaphoreType.DMA,
            pltpu.SemaphoreType.DMA,
            pltpu.SemaphoreType.DMA,
        ],
        compiler_params=_cp,
    )
    def agg_kernel(z_hbm, edges_hbm, out_hbm,
                   src_v, dst_v, z_v, acc_v, sem_s, sem_d, sem_z):
        wid = lax.axis_index("s") * 2 + lax.axis_index("c")
        cp_s = pltpu.async_copy(edges_hbm.at[pl.ds(wid * 2 * ep, ep)],
                                src_v, sem_s)
        cp_d = pltpu.async_copy(edges_hbm.at[pl.ds(wid * 2 * ep + ep, ep)],
                                dst_v, sem_d)
        cp_z = pltpu.async_copy(z_hbm, z_v, sem_z)
        _zero_fill(acc_v, tbl, 10)
        cp_s.wait()
        cp_d.wait()
        cp_z.wait()

        def do_group(s, d):
            vals = []
            for c in range(w):
                si = (s + c * n_nodes) if c else s
                vals.append((plsc.load_gather(z_v, [si]),
                             (d + c * n_nodes) if c else d))
            for v, di in vals:
                plsc.addupdate_scatter(acc_v, [di], v)

        @pl.loop(0, ep_main, step=2 * _LANES)
        def _(i):
            s0 = src_v[pl.ds(i, _LANES)]
            d0 = dst_v[pl.ds(i, _LANES)]
            s1 = src_v[pl.ds(i + _LANES, _LANES)]
            d1 = dst_v[pl.ds(i + _LANES, _LANES)]
            do_group(s0, d0)
            do_group(s1, d1)

        for i in range(ep_main, ep, _LANES):
            do_group(src_v[pl.ds(i, _LANES)], dst_v[pl.ds(i, _LANES)])

        pltpu.sync_copy(acc_v, out_hbm.at[pl.ds(wid * tblp, tbl)])

    return agg_kernel


def _rows2d(flat, w, n):
    return jnp.concatenate([flat[c * n:(c + 1) * n][None, :] for c in range(w)],
                           axis=0)


def _sum_parts(parts_ref, span):
    stride = _pad128(span)
    acc = parts_ref[pl.ds(0, span)]
    for i in range(1, NUM_WORKERS):
        acc = acc + parts_ref[pl.ds(i * stride, span)]
    return acc


def _tc0_body(x_ref, w1t_ref, xw1_ref):
    n = x_ref.shape[0]
    w1 = w1t_ref.shape[0]
    xwt = lax.dot_general(w1t_ref[...], x_ref[...],
                          (((1,), (1,)), ((), ())),
                          precision=_PREC,
                          preferred_element_type=jnp.float32)
    for c in range(w1):
        xw1_ref[pl.ds(c * n, n)] = xwt[c]


def _tc1_body(deg_parts_ref, xw1_ref, dinv_ref, z1_ref):
    n = dinv_ref.shape[0]
    w1 = xw1_ref.shape[0] // n
    deg = _sum_parts(deg_parts_ref, n) + 1.0
    dinv = lax.rsqrt(deg)
    dinv_ref[...] = dinv
    for c in range(w1):
        z1_ref[pl.ds(c * n, n)] = xw1_ref[pl.ds(c * n, n)] * dinv


def _tc_layer_body(parts_ref, z_ref, dinv_ref, wt_ref, bcol_ref, zout_ref):
    w_out, w = wt_ref.shape
    n = dinv_ref.shape[0]
    flat = _sum_parts(parts_ref, w * n)
    acc = _rows2d(flat + z_ref[...], w, n)
    dinv = dinv_ref[...][None, :]
    h = jnp.maximum(dinv * acc + bcol_ref[...], 0.0)
    zout = lax.dot_general(wt_ref[...], h,
                           (((1,), (0,)), ((), ())),
                           precision=_PREC_SMALL,
                           preferred_element_type=jnp.float32) * dinv
    for c in range(w_out):
        zout_ref[pl.ds(c * n, n)] = zout[c]


def _tc_final_body(parts_ref, z_ref, dinv_ref, wcx_ref, b3col_ref, bcrow_ref,
                   h_ref, out_ref):
    w = b3col_ref.shape[0]
    n = dinv_ref.shape[0]
    flat = _sum_parts(parts_ref, w * n)
    acc = _rows2d(flat + z_ref[...], w, n)
    dinv = dinv_ref[...][None, :]
    h = jnp.maximum(dinv * acc + b3col_ref[...], 0.0)
    both = lax.dot_general(h, wcx_ref[...],
                           (((0,), (0,)), ((), ())),
                           precision=_PREC_SMALL,
                           preferred_element_type=jnp.float32)
    h_ref[...] = both[:, :w]
    out_ref[...] = both[:, w:] + bcrow_ref[...]


def _f32(shape):
    return jax.ShapeDtypeStruct(shape, jnp.float32)


def kernel(x, edge_index, W1, b1, W2, b2, W3, b3, Wc, bc):
    n, _ = x.shape
    e = edge_index.shape[1]
    w1 = W1.shape[1]
    w2 = W2.shape[1]
    w3 = W3.shape[1]
    c_out = Wc.shape[1]

    ei = edge_index.astype(jnp.int32)

    deg_parts, edges = _make_deg_kernel(n, e)(ei)

    xw1 = pl.pallas_call(_tc0_body, out_shape=_f32((w1 * n,)))(x, W1.T)

    dinv, z1 = pl.pallas_call(
        _tc1_body,
        out_shape=[_f32((n,)), _f32((w1 * n,))],
    )(deg_parts, xw1)

    p1 = _make_agg_kernel(n, e, w1)(z1, edges)
    z2 = pl.pallas_call(
        _tc_layer_body,
        out_shape=_f32((w2 * n,)),
    )(p1, z1, dinv, W2.T, b1[:, None])

    p2 = _make_agg_kernel(n, e, w2)(z2, edges)
    z3 = pl.pallas_call(
        _tc_layer_body,
        out_shape=_f32((w3 * n,)),
    )(p2, z2, dinv, W3.T, b2[:, None])

    p3 = _make_agg_kernel(n, e, w3)(z3, edges)
    wcx = jnp.concatenate([jnp.eye(w3, dtype=jnp.float32), Wc], axis=1)
    h, out = pl.pallas_call(
        _tc_final_body,
        out_shape=[_f32((n, w3)), _f32((n, c_out))],
    )(p3, z3, dinv, wcx, b3[:, None], bc[None, :])

    return (out, h)
